# baseline (device time: 12500 ns/iter reference)
import jax
import jax.numpy as jnp
from jax import lax
from jax.experimental import pallas as pl
from jax.experimental.pallas import tpu as pltpu

M = 512
N = 1024
HALF = N // 2
NCHUNK = 4
CROWS = M // NCHUNK


def kernel(x):
    def body(
        x_hbm,
        out_hbm,
        xv,
        send_buf,
        recv_buf,
        outv,
        in_sems,
        out_sems,
        send_sems,
        recv_sems,
    ):
        my_x = lax.axis_index("x")
        my_y = lax.axis_index("y")
        my_z = lax.axis_index("z")
        other_x = 1 - my_x

        def in_dma(c):
            rows = pl.ds(c * CROWS, CROWS)
            return pltpu.make_async_copy(
                x_hbm.at[0, rows, :], xv.at[rows], in_sems.at[c]
            )

        def out_dma(c):
            rows = pl.ds(c * CROWS, CROWS)
            return pltpu.make_async_copy(
                outv.at[rows], out_hbm.at[rows], out_sems.at[c]
            )

        def rdma(c):
            rows = pl.ds(c * CROWS, CROWS)
            return pltpu.make_async_remote_copy(
                src_ref=send_buf.at[rows],
                dst_ref=recv_buf.at[rows],
                send_sem=send_sems.at[c],
                recv_sem=recv_sems.at[c],
                device_id=(other_x, my_y, my_z),
                device_id_type=pl.DeviceIdType.MESH,
            )

        for c in range(NCHUNK):
            in_dma(c).start()

        barrier_sem = pltpu.get_barrier_semaphore()
        pl.semaphore_signal(
            barrier_sem,
            inc=1,
            device_id=(other_x, my_y, my_z),
            device_id_type=pl.DeviceIdType.MESH,
        )
        pl.semaphore_wait(barrier_sem, 1)

        for c in range(NCHUNK):
            rows = pl.ds(c * CROWS, CROWS)
            in_dma(c).wait()

            @pl.when(my_x == 0)
            def _():
                send_buf[rows] = xv[rows, HALF:].astype(jnp.bfloat16)

            @pl.when(my_x == 1)
            def _():
                send_buf[rows] = xv[rows, :HALF].astype(jnp.bfloat16)

            rdma(c).start()

        for c in range(NCHUNK):
            rows = pl.ds(c * CROWS, CROWS)
            rdma(c).wait_recv()

            @pl.when(my_x == 0)
            def _():
                outv[rows] = xv[rows, :HALF] + recv_buf[rows].astype(jnp.float32)

            @pl.when(my_x == 1)
            def _():
                outv[rows] = xv[rows, HALF:] + recv_buf[rows].astype(jnp.float32)

            out_dma(c).start()

        for c in range(NCHUNK):
            out_dma(c).wait()
            rdma(c).wait_send()

    return pl.pallas_call(
        body,
        out_shape=jax.ShapeDtypeStruct((M, HALF), jnp.float32),
        in_specs=[pl.BlockSpec(memory_space=pl.ANY)],
        out_specs=pl.BlockSpec(memory_space=pl.ANY),
        scratch_shapes=[
            pltpu.VMEM((M, N), jnp.float32),
            pltpu.VMEM((M, HALF), jnp.bfloat16),
            pltpu.VMEM((M, HALF), jnp.bfloat16),
            pltpu.VMEM((M, HALF), jnp.float32),
            pltpu.SemaphoreType.DMA((NCHUNK,)),
            pltpu.SemaphoreType.DMA((NCHUNK,)),
            pltpu.SemaphoreType.DMA((NCHUNK,)),
            pltpu.SemaphoreType.DMA((NCHUNK,)),
        ],
        compiler_params=pltpu.CompilerParams(collective_id=0),
    )(x)
